# baseline (speedup 1.0000x reference)
import functools

import jax
import jax.numpy as jnp
from jax import lax
from jax.experimental import pallas as pl
from jax.experimental.pallas import tpu as pltpu
from jax.experimental.pallas import tpu_sc as plsc

VOCAB = 400000
EMB_DIM = 300
D_PAD = 16
ROW_BLK = 8000
NC, NS = 2, 16
NW = NC * NS
B_TOT = 4096 * 50
B_PER_W = B_TOT // NW


def _project_body(t_ref, w_ref, b_ref, o_ref):
    o_ref[...] = (
        jnp.dot(t_ref[...], w_ref[...], preferred_element_type=jnp.float32)
        + b_ref[...]
    )


def _project(table, w_pad, b_pad):
    grid = (VOCAB // ROW_BLK,)
    return pl.pallas_call(
        _project_body,
        grid=grid,
        in_specs=[
            pl.BlockSpec((ROW_BLK, EMB_DIM), lambda i: (i, 0)),
            pl.BlockSpec((EMB_DIM, D_PAD), lambda i: (0, 0)),
            pl.BlockSpec((1, D_PAD), lambda i: (0, 0)),
        ],
        out_specs=pl.BlockSpec((ROW_BLK, D_PAD), lambda i: (i, 0)),
        out_shape=jax.ShapeDtypeStruct((VOCAB, D_PAD), jnp.float32),
    )(table, w_pad, b_pad)


def _gather_body(adj_hbm, p_hbm, out_hbm, idx_v, rows_v, sem):
    wid = lax.axis_index("s") * NC + lax.axis_index("c")
    pltpu.sync_copy(adj_hbm.at[wid], idx_v)
    pltpu.async_copy(p_hbm.at[idx_v], rows_v, sem).wait()
    pltpu.sync_copy(rows_v, out_hbm.at[wid])


_gather = functools.partial(
    pl.kernel,
    mesh=plsc.VectorSubcoreMesh(
        core_axis_name="c", subcore_axis_name="s", num_cores=NC, num_subcores=NS
    ),
    out_type=jax.ShapeDtypeStruct((NW, B_PER_W, D_PAD), jnp.float32),
    scratch_types=[
        pltpu.VMEM((B_PER_W,), jnp.int32),
        pltpu.VMEM((B_PER_W, D_PAD), jnp.float32),
        pltpu.SemaphoreType.DMA,
    ],
    compiler_params=pltpu.CompilerParams(use_tc_tiling_on_sc=False),
)(_gather_body)


def kernel(adj, table, W, b):
    w_pad = jnp.pad(W, ((0, 0), (0, D_PAD - W.shape[1])))
    b_pad = jnp.pad(b, (0, D_PAD - b.shape[0])).reshape(1, D_PAD)
    proj = _project(table, w_pad, b_pad)
    adj_w = adj.reshape(NW, B_PER_W).astype(jnp.int32)
    out = _gather(adj_w, proj)
    return out.reshape(4096, 50, D_PAD)[..., : b.shape[0]]
